# baseline (speedup 1.0000x reference)
import jax
import jax.numpy as jnp
from jax.experimental import pallas as pl

C_CHUNK = 2048
G_EXTRA = 2048
S_TOTAL = 2 * C_CHUNK + G_EXTRA

BLOCK_S = 256
N_BLOCKS = S_TOTAL // BLOCK_S
N_COPY = C_CHUNK // BLOCK_S


def _body(k_ref, v_ref, fk_ref, K_ref, V_ref, FK_ref):
    j = pl.program_id(1)

    @pl.when(j < N_COPY)
    def _copy():
        K_ref[...] = k_ref[...]
        V_ref[...] = v_ref[...]
        FK_ref[...] = fk_ref[...]

    @pl.when(j >= N_COPY)
    def _zero():
        K_ref[...] = jnp.zeros(K_ref.shape, K_ref.dtype)
        V_ref[...] = jnp.zeros(V_ref.shape, V_ref.dtype)
        FK_ref[...] = jnp.zeros(FK_ref.shape, FK_ref.dtype)


def kernel(k_c, v_c, fk_c):
    B, C, H, D = k_c.shape
    F = fk_c.shape[-1]

    def in_map(b, j):
        return (b, jnp.minimum(j, N_COPY - 1), 0, 0)

    def out_map(b, j):
        return (b, j, 0, 0)

    K, V, FK = pl.pallas_call(
        _body,
        grid=(B, N_BLOCKS),
        in_specs=[
            pl.BlockSpec((1, BLOCK_S, H, D), in_map),
            pl.BlockSpec((1, BLOCK_S, H, D), in_map),
            pl.BlockSpec((1, BLOCK_S, H, F), in_map),
        ],
        out_specs=[
            pl.BlockSpec((1, BLOCK_S, H, D), out_map),
            pl.BlockSpec((1, BLOCK_S, H, D), out_map),
            pl.BlockSpec((1, BLOCK_S, H, F), out_map),
        ],
        out_shape=[
            jax.ShapeDtypeStruct((B, S_TOTAL, H, D), k_c.dtype),
            jax.ShapeDtypeStruct((B, S_TOTAL, H, D), v_c.dtype),
            jax.ShapeDtypeStruct((B, S_TOTAL, H, F), fk_c.dtype),
        ],
    )(k_c, v_c, fk_c)

    Hs = jnp.zeros((B, H, F, D), dtype=k_c.dtype)
    S = jnp.zeros((B, H, F), dtype=k_c.dtype)
    return (K, V, FK, Hs, S)
